# baseline (speedup 1.0000x reference)
import jax
import jax.numpy as jnp
from jax.experimental import pallas as pl

N = 10000
E = 160000
B = 64
T = 6
DIN = 768
H = 256
SEQ_IN = 128
NC = 14
R = 4
L = 3
ALPHA = 0.15
BM = 1000

_pc = pl.pallas_call


def _mm_body(x_ref, w_ref, b_ref, o_ref):
    o_ref[...] = jnp.dot(x_ref[...], w_ref[...],
                         preferred_element_type=jnp.float32) + b_ref[...]


def _dense(xa, w, b, bm=BM):
    m, k = xa.shape
    n = w.shape[1]
    return _pc(
        _mm_body,
        grid=(m // bm,),
        in_specs=[pl.BlockSpec((bm, k), lambda i: (i, 0)),
                  pl.BlockSpec((k, n), lambda i: (0, 0)),
                  pl.BlockSpec((1, n), lambda i: (0, 0))],
        out_specs=pl.BlockSpec((bm, n), lambda i: (i, 0)),
        out_shape=jax.ShapeDtypeStruct((m, n), jnp.float32),
    )(xa, w, b.reshape(1, n))


def _upd_body(agg_ref, sw_ref, h0_ref, g_ref, bz_ref, o_ref):
    out = (1.0 - ALPHA) * agg_ref[...] + sw_ref[...] + ALPHA * h0_ref[...]
    mu = jnp.mean(out, axis=1, keepdims=True)
    var = jnp.mean((out - mu) ** 2, axis=1, keepdims=True)
    out = (out - mu) / jnp.sqrt(var + 1e-5) * g_ref[...] + bz_ref[...]
    o_ref[...] = 0.5 * out * (1.0 + jax.lax.erf(out * 0.7071067811865475))


def _update(agg, sw, h0, g, bz):
    return _pc(
        _upd_body,
        grid=(N // BM,),
        in_specs=[pl.BlockSpec((BM, H), lambda i: (i, 0)),
                  pl.BlockSpec((BM, H), lambda i: (i, 0)),
                  pl.BlockSpec((BM, H), lambda i: (i, 0)),
                  pl.BlockSpec((1, H), lambda i: (0, 0)),
                  pl.BlockSpec((1, H), lambda i: (0, 0))],
        out_specs=pl.BlockSpec((BM, H), lambda i: (i, 0)),
        out_shape=jax.ShapeDtypeStruct((N, H), jnp.float32),
    )(agg, sw, h0, g.reshape(1, H), bz.reshape(1, H))


def _gru_step(xt, h, wih_t, whh_t, bih, bhh):
    gi = jnp.dot(xt, wih_t, preferred_element_type=jnp.float32) + bih
    gh = jnp.dot(h, whh_t, preferred_element_type=jnp.float32) + bhh
    r = jax.nn.sigmoid(gi[:, :H] + gh[:, :H])
    z = jax.nn.sigmoid(gi[:, H:2 * H] + gh[:, H:2 * H])
    n = jnp.tanh(gi[:, 2 * H:] + r * gh[:, 2 * H:])
    return (1.0 - z) * n + z * h


def _seq_body(xs_ref, wihf_ref, whhf_ref, bihf_ref, bhhf_ref,
              wihb_ref, whhb_ref, bihb_ref, bhhb_ref,
              s1_ref, sb1_ref, s2_ref, sb2_ref, s3_ref, sb3_ref, o_ref):
    wihf = wihf_ref[...]
    whhf = whhf_ref[...]
    bihf = bihf_ref[...]
    bhhf = bhhf_ref[...]
    wihb = wihb_ref[...]
    whhb = whhb_ref[...]
    bihb = bihb_ref[...]
    bhhb = bhhb_ref[...]
    hf = jnp.zeros((B, H), dtype=jnp.float32)
    hb = jnp.zeros((B, H), dtype=jnp.float32)
    fs = [None] * T
    bs_ = [None] * T
    for i in range(T):
        hf = _gru_step(xs_ref[i], hf, wihf, whhf, bihf, bhhf)
        fs[i] = hf
        hb = _gru_step(xs_ref[T - 1 - i], hb, wihb, whhb, bihb, bhhb)
        bs_[T - 1 - i] = hb
    acc = jnp.zeros((B, 2 * H), dtype=jnp.float32)
    mx = jnp.full((B, 2 * H), -jnp.inf, dtype=jnp.float32)
    for t in range(T):
        so_t = jnp.concatenate([fs[t], bs_[t]], axis=1)
        acc = acc + so_t
        mx = jnp.maximum(mx, so_t)
    sf = acc * (1.0 / T) + mx
    sf = jnp.maximum(jnp.dot(sf, s1_ref[...], preferred_element_type=jnp.float32)
                     + sb1_ref[...], 0.0)
    sf = jnp.maximum(jnp.dot(sf, s2_ref[...], preferred_element_type=jnp.float32)
                     + sb2_ref[...], 0.0)
    o_ref[...] = jnp.dot(sf, s3_ref[...], preferred_element_type=jnp.float32) + sb3_ref[...]


def _seq_branch(sequence, gru_Wih, gru_Whh, gru_bih, gru_bhh,
                S1, Sb1, S2, Sb2, S3, Sb3):
    xs = jnp.transpose(sequence.reshape(B, T, SEQ_IN), (1, 0, 2))
    args = (xs,
            gru_Wih[0].T, gru_Whh[0].T, gru_bih[0].reshape(1, 3 * H),
            gru_bhh[0].reshape(1, 3 * H),
            gru_Wih[1].T, gru_Whh[1].T, gru_bih[1].reshape(1, 3 * H),
            gru_bhh[1].reshape(1, 3 * H),
            S1, Sb1.reshape(1, H), S2, Sb2.reshape(1, H // 2),
            S3, Sb3.reshape(1, NC))
    specs = [pl.BlockSpec(a.shape, lambda i, _n=len(a.shape): (0,) * _n)
             for a in args]
    return _pc(
        _seq_body,
        grid=(1,),
        in_specs=specs,
        out_specs=pl.BlockSpec((B, NC), lambda i: (0, 0)),
        out_shape=jax.ShapeDtypeStruct((B, NC), jnp.float32),
    )(*args)


def _pool_body(h_ref, bt_ref, g1_ref, gb1_ref, g2_ref, gb2_ref,
               g3_ref, gb3_ref, sl_ref, o_ref):
    iota_b = jax.lax.broadcasted_iota(jnp.int32, (B, N), 0)
    oh = (iota_b == bt_ref[...]).astype(jnp.float32)
    pooled = jnp.dot(oh, h_ref[...], preferred_element_type=jnp.float32)
    cnt = jnp.sum(oh, axis=1, keepdims=True)
    pooled = pooled / jnp.maximum(cnt, 1.0)
    gf = jnp.maximum(jnp.dot(pooled, g1_ref[...], preferred_element_type=jnp.float32)
                     + gb1_ref[...], 0.0)
    gf = jnp.maximum(jnp.dot(gf, g2_ref[...], preferred_element_type=jnp.float32)
                     + gb2_ref[...], 0.0)
    o_ref[...] = (jnp.dot(gf, g3_ref[...], preferred_element_type=jnp.float32)
                  + gb3_ref[...] + sl_ref[...])


def _pool_head(h, batch, G1, Gb1, G2, Gb2, G3, Gb3, seq_logits):
    args = (h, batch.reshape(1, N).astype(jnp.int32),
            G1, Gb1.reshape(1, H // 2), G2, Gb2.reshape(1, H // 4),
            G3, Gb3.reshape(1, NC), seq_logits)
    specs = [pl.BlockSpec(a.shape, lambda i, _n=len(a.shape): (0,) * _n)
             for a in args]
    return _pc(
        _pool_body,
        grid=(1,),
        in_specs=specs,
        out_specs=pl.BlockSpec((B, NC), lambda i: (0, 0)),
        out_shape=jax.ShapeDtypeStruct((B, NC), jnp.float32),
    )(*args)


def kernel(x, edge_index, edge_type, sequence, batch, Wp, bp, Wr, Ws, bs,
           A1, b1, A2, b2, Eemb, imp, ln_g, ln_b,
           gru_Wih, gru_Whh, gru_bih, gru_bhh,
           G1, Gb1, G2, Gb2, G3, Gb3, S1, Sb1, S2, Sb2, S3, Sb3):
    src = edge_index[0]
    dst = edge_index[1]
    et = edge_type.astype(jnp.int32)

    seq_logits = _seq_branch(sequence, gru_Wih, gru_Whh, gru_bih, gru_bhh,
                             S1, Sb1, S2, Sb2, S3, Sb3)

    h = _dense(x, Wp, bp)
    h0 = h

    ew_all = jax.nn.softmax(imp, axis=-1)
    ce_all = jnp.mean(Eemb, axis=2)

    for l in range(L):
        wcat = jnp.concatenate([
            jnp.transpose(Wr[l], (1, 0, 2)).reshape(H, R * H),
            jnp.transpose(A1[l, :, :H, :], (1, 0, 2)).reshape(H, R * (H // 2)),
            jnp.transpose(A1[l, :, H:, :], (1, 0, 2)).reshape(H, R * (H // 2)),
            Ws[l],
        ], axis=1)
        bcat = jnp.concatenate([
            jnp.zeros((R * H,), jnp.float32),
            b1[l].reshape(R * (H // 2)),
            jnp.zeros((R * (H // 2),), jnp.float32),
            bs[l],
        ])
        o = _dense(h, wcat, bcat)
        txf = o[:, :R * H].reshape(N * R, H)
        hdf = o[:, R * H:R * H + R * (H // 2)].reshape(N * R, H // 2)
        hsf = o[:, R * H + R * (H // 2):R * H + R * H].reshape(N * R, H // 2)
        sw = o[:, R * H + R * H:]

        idx_d = dst * R + et
        idx_s = src * R + et
        g_d = jax.lax.optimization_barrier(hdf[idx_d])
        g_s = jax.lax.optimization_barrier(hsf[idx_s])
        pre = jnp.maximum(g_d + g_s, 0.0)
        a2 = A2[l, :, :, 0]
        sc = jnp.sum(pre * a2[et], axis=1) + b2[l, et, 0]
        sc = jnp.where(sc >= 0.0, sc, 0.2 * sc) + ce_all[l, et]
        mxs = jnp.full((R,), -1e30, jnp.float32).at[et].max(sc)
        es = jnp.exp(sc - mxs[et])
        dn = jnp.zeros((R,), jnp.float32).at[et].add(es)
        w = es / dn[et] * ew_all[l, et]
        msg = w[:, None] * jax.lax.optimization_barrier(txf[idx_s])
        agg = jnp.zeros((N, H), jnp.float32).at[dst].add(msg)

        h = _update(agg, sw, h0, ln_g[l], ln_b[l])

    return _pool_head(h, batch, G1, Gb1, G2, Gb2, G3, Gb3, seq_logits)


# speedup vs baseline: 5.2703x; 5.2703x over previous
import functools

import jax
import jax.numpy as jnp
from jax import lax
from jax.experimental import pallas as pl
from jax.experimental.pallas import tpu as pltpu
from jax.experimental.pallas import tpu_sc as plsc

N = 10000
E = 160000
B = 64
T = 6
DIN = 768
H = 256
SEQ_IN = 128
NC = 14
R = 4
L = 3
ALPHA = 0.15
BM = 1000

_pc = pl.pallas_call


_SC_CH = 128


def _sc_gather_body(nchunks, tail, d, table_hbm, idx_hbm, out_hbm,
                    idx_v, rows_v, idx_t, rows_t, sem):
    info = plsc.get_sparse_core_info()
    nw = info.num_cores * info.num_subcores
    span = idx_hbm.shape[0] // nw
    wid = lax.axis_index("s") * info.num_cores + lax.axis_index("c")
    base = wid * span

    def chunk(j, _):
        off = base + j * _SC_CH
        pltpu.sync_copy(idx_hbm.at[pl.ds(off, _SC_CH)], idx_v)
        pltpu.async_copy(table_hbm.at[idx_v], rows_v, sem).wait()
        pltpu.sync_copy(rows_v, out_hbm.at[pl.ds(off, _SC_CH)])
        return ()

    lax.fori_loop(0, nchunks, chunk, ())
    if tail:
        off = base + nchunks * _SC_CH
        pltpu.sync_copy(idx_hbm.at[pl.ds(off, tail)], idx_t)
        pltpu.async_copy(table_hbm.at[idx_t], rows_t, sem).wait()
        pltpu.sync_copy(rows_t, out_hbm.at[pl.ds(off, tail)])


def _sc_gather(table, idx):
    v, d = table.shape
    e = idx.shape[0]
    info = plsc.get_sparse_core_info()
    nw = info.num_cores * info.num_subcores
    span = e // nw
    assert span * nw == e and span % 8 == 0
    nchunks = span // _SC_CH
    tail = span - nchunks * _SC_CH
    mesh = plsc.VectorSubcoreMesh(core_axis_name="c", subcore_axis_name="s")
    fn = functools.partial(_sc_gather_body, nchunks, tail, d)
    k = pl.kernel(
        fn,
        mesh=mesh,
        out_type=jax.ShapeDtypeStruct((e, d), jnp.float32),
        scratch_types=[
            pltpu.VMEM((_SC_CH,), jnp.int32),
            pltpu.VMEM((_SC_CH, d), jnp.float32),
            pltpu.VMEM((max(tail, 8),), jnp.int32),
            pltpu.VMEM((max(tail, 8), d), jnp.float32),
            pltpu.SemaphoreType.DMA,
        ],
    )
    return k(table, idx.astype(jnp.int32))


def _mm_body(x_ref, w_ref, b_ref, o_ref):
    o_ref[...] = jnp.dot(x_ref[...], w_ref[...],
                         preferred_element_type=jnp.float32) + b_ref[...]


def _dense(xa, w, b, bm=BM):
    m, k = xa.shape
    n = w.shape[1]
    return _pc(
        _mm_body,
        grid=(m // bm,),
        in_specs=[pl.BlockSpec((bm, k), lambda i: (i, 0)),
                  pl.BlockSpec((k, n), lambda i: (0, 0)),
                  pl.BlockSpec((1, n), lambda i: (0, 0))],
        out_specs=pl.BlockSpec((bm, n), lambda i: (i, 0)),
        out_shape=jax.ShapeDtypeStruct((m, n), jnp.float32),
    )(xa, w, b.reshape(1, n))


def _upd_body(agg_ref, sw_ref, h0_ref, g_ref, bz_ref, o_ref):
    out = (1.0 - ALPHA) * agg_ref[...] + sw_ref[...] + ALPHA * h0_ref[...]
    mu = jnp.mean(out, axis=1, keepdims=True)
    var = jnp.mean((out - mu) ** 2, axis=1, keepdims=True)
    out = (out - mu) / jnp.sqrt(var + 1e-5) * g_ref[...] + bz_ref[...]
    o_ref[...] = 0.5 * out * (1.0 + jax.lax.erf(out * 0.7071067811865475))


def _update(agg, sw, h0, g, bz):
    return _pc(
        _upd_body,
        grid=(N // BM,),
        in_specs=[pl.BlockSpec((BM, H), lambda i: (i, 0)),
                  pl.BlockSpec((BM, H), lambda i: (i, 0)),
                  pl.BlockSpec((BM, H), lambda i: (i, 0)),
                  pl.BlockSpec((1, H), lambda i: (0, 0)),
                  pl.BlockSpec((1, H), lambda i: (0, 0))],
        out_specs=pl.BlockSpec((BM, H), lambda i: (i, 0)),
        out_shape=jax.ShapeDtypeStruct((N, H), jnp.float32),
    )(agg, sw, h0, g.reshape(1, H), bz.reshape(1, H))


def _gru_step(xt, h, wih_t, whh_t, bih, bhh):
    gi = jnp.dot(xt, wih_t, preferred_element_type=jnp.float32) + bih
    gh = jnp.dot(h, whh_t, preferred_element_type=jnp.float32) + bhh
    r = jax.nn.sigmoid(gi[:, :H] + gh[:, :H])
    z = jax.nn.sigmoid(gi[:, H:2 * H] + gh[:, H:2 * H])
    n = jnp.tanh(gi[:, 2 * H:] + r * gh[:, 2 * H:])
    return (1.0 - z) * n + z * h


def _seq_body(xs_ref, wihf_ref, whhf_ref, bihf_ref, bhhf_ref,
              wihb_ref, whhb_ref, bihb_ref, bhhb_ref,
              s1_ref, sb1_ref, s2_ref, sb2_ref, s3_ref, sb3_ref, o_ref):
    wihf = wihf_ref[...]
    whhf = whhf_ref[...]
    bihf = bihf_ref[...]
    bhhf = bhhf_ref[...]
    wihb = wihb_ref[...]
    whhb = whhb_ref[...]
    bihb = bihb_ref[...]
    bhhb = bhhb_ref[...]
    hf = jnp.zeros((B, H), dtype=jnp.float32)
    hb = jnp.zeros((B, H), dtype=jnp.float32)
    fs = [None] * T
    bs_ = [None] * T
    for i in range(T):
        hf = _gru_step(xs_ref[i], hf, wihf, whhf, bihf, bhhf)
        fs[i] = hf
        hb = _gru_step(xs_ref[T - 1 - i], hb, wihb, whhb, bihb, bhhb)
        bs_[T - 1 - i] = hb
    acc = jnp.zeros((B, 2 * H), dtype=jnp.float32)
    mx = jnp.full((B, 2 * H), -jnp.inf, dtype=jnp.float32)
    for t in range(T):
        so_t = jnp.concatenate([fs[t], bs_[t]], axis=1)
        acc = acc + so_t
        mx = jnp.maximum(mx, so_t)
    sf = acc * (1.0 / T) + mx
    sf = jnp.maximum(jnp.dot(sf, s1_ref[...], preferred_element_type=jnp.float32)
                     + sb1_ref[...], 0.0)
    sf = jnp.maximum(jnp.dot(sf, s2_ref[...], preferred_element_type=jnp.float32)
                     + sb2_ref[...], 0.0)
    o_ref[...] = jnp.dot(sf, s3_ref[...], preferred_element_type=jnp.float32) + sb3_ref[...]


def _seq_branch(sequence, gru_Wih, gru_Whh, gru_bih, gru_bhh,
                S1, Sb1, S2, Sb2, S3, Sb3):
    xs = jnp.transpose(sequence.reshape(B, T, SEQ_IN), (1, 0, 2))
    args = (xs,
            gru_Wih[0].T, gru_Whh[0].T, gru_bih[0].reshape(1, 3 * H),
            gru_bhh[0].reshape(1, 3 * H),
            gru_Wih[1].T, gru_Whh[1].T, gru_bih[1].reshape(1, 3 * H),
            gru_bhh[1].reshape(1, 3 * H),
            S1, Sb1.reshape(1, H), S2, Sb2.reshape(1, H // 2),
            S3, Sb3.reshape(1, NC))
    specs = [pl.BlockSpec(a.shape, lambda i, _n=len(a.shape): (0,) * _n)
             for a in args]
    return _pc(
        _seq_body,
        grid=(1,),
        in_specs=specs,
        out_specs=pl.BlockSpec((B, NC), lambda i: (0, 0)),
        out_shape=jax.ShapeDtypeStruct((B, NC), jnp.float32),
    )(*args)


def _pool_body(h_ref, bt_ref, g1_ref, gb1_ref, g2_ref, gb2_ref,
               g3_ref, gb3_ref, sl_ref, o_ref):
    iota_b = jax.lax.broadcasted_iota(jnp.int32, (B, N), 0)
    oh = (iota_b == bt_ref[...]).astype(jnp.float32)
    pooled = jnp.dot(oh, h_ref[...], preferred_element_type=jnp.float32)
    cnt = jnp.sum(oh, axis=1, keepdims=True)
    pooled = pooled / jnp.maximum(cnt, 1.0)
    gf = jnp.maximum(jnp.dot(pooled, g1_ref[...], preferred_element_type=jnp.float32)
                     + gb1_ref[...], 0.0)
    gf = jnp.maximum(jnp.dot(gf, g2_ref[...], preferred_element_type=jnp.float32)
                     + gb2_ref[...], 0.0)
    o_ref[...] = (jnp.dot(gf, g3_ref[...], preferred_element_type=jnp.float32)
                  + gb3_ref[...] + sl_ref[...])


def _pool_head(h, batch, G1, Gb1, G2, Gb2, G3, Gb3, seq_logits):
    args = (h, batch.reshape(1, N).astype(jnp.int32),
            G1, Gb1.reshape(1, H // 2), G2, Gb2.reshape(1, H // 4),
            G3, Gb3.reshape(1, NC), seq_logits)
    specs = [pl.BlockSpec(a.shape, lambda i, _n=len(a.shape): (0,) * _n)
             for a in args]
    return _pc(
        _pool_body,
        grid=(1,),
        in_specs=specs,
        out_specs=pl.BlockSpec((B, NC), lambda i: (0, 0)),
        out_shape=jax.ShapeDtypeStruct((B, NC), jnp.float32),
    )(*args)


def kernel(x, edge_index, edge_type, sequence, batch, Wp, bp, Wr, Ws, bs,
           A1, b1, A2, b2, Eemb, imp, ln_g, ln_b,
           gru_Wih, gru_Whh, gru_bih, gru_bhh,
           G1, Gb1, G2, Gb2, G3, Gb3, S1, Sb1, S2, Sb2, S3, Sb3):
    src = edge_index[0]
    dst = edge_index[1]
    et = edge_type.astype(jnp.int32)

    seq_logits = _seq_branch(sequence, gru_Wih, gru_Whh, gru_bih, gru_bhh,
                             S1, Sb1, S2, Sb2, S3, Sb3)

    h = _dense(x, Wp, bp)
    h0 = h

    ew_all = jax.nn.softmax(imp, axis=-1)
    ce_all = jnp.mean(Eemb, axis=2)

    for l in range(L):
        wcat = jnp.concatenate([
            jnp.transpose(Wr[l], (1, 0, 2)).reshape(H, R * H),
            jnp.transpose(A1[l, :, :H, :], (1, 0, 2)).reshape(H, R * (H // 2)),
            jnp.transpose(A1[l, :, H:, :], (1, 0, 2)).reshape(H, R * (H // 2)),
            Ws[l],
        ], axis=1)
        bcat = jnp.concatenate([
            jnp.zeros((R * H,), jnp.float32),
            b1[l].reshape(R * (H // 2)),
            jnp.zeros((R * (H // 2),), jnp.float32),
            bs[l],
        ])
        o = _dense(h, wcat, bcat)
        txf = o[:, :R * H].reshape(N * R, H)
        hdf = o[:, R * H:R * H + R * (H // 2)].reshape(N * R, H // 2)
        hsf = o[:, R * H + R * (H // 2):R * H + R * H].reshape(N * R, H // 2)
        sw = o[:, R * H + R * H:]

        idx_d = dst * R + et
        idx_s = src * R + et
        g_d = _sc_gather(hdf, idx_d)
        g_s = _sc_gather(hsf, idx_s)
        pre = jnp.maximum(g_d + g_s, 0.0)
        oh = (et[:, None] == jnp.arange(R, dtype=jnp.int32)).astype(jnp.float32)
        a2e = oh @ A2[l, :, :, 0]
        sc = jnp.sum(pre * a2e, axis=1) + oh @ b2[l, :, 0]
        sc = jnp.where(sc >= 0.0, sc, 0.2 * sc) + oh @ ce_all[l]
        mxs = jnp.max(jnp.where(oh > 0.0, sc[:, None], -1e30), axis=0)
        es = jnp.exp(sc - oh @ mxs)
        dn = es @ oh
        w = es / (oh @ dn) * (oh @ ew_all[l])
        msg = w[:, None] * _sc_gather(txf, idx_s)
        agg = jnp.zeros((N, H), jnp.float32).at[dst].add(msg)

        h = _update(agg, sw, h0, ln_g[l], ln_b[l])

    return _pool_head(h, batch, G1, Gb1, G2, Gb2, G3, Gb3, seq_logits)


# speedup vs baseline: 5.3949x; 1.0236x over previous
import functools

import jax
import jax.numpy as jnp
from jax import lax
from jax.experimental import pallas as pl
from jax.experimental.pallas import tpu as pltpu
from jax.experimental.pallas import tpu_sc as plsc

N = 10000
E = 160000
B = 64
T = 6
DIN = 768
H = 256
SEQ_IN = 128
NC = 14
R = 4
L = 3
ALPHA = 0.15
BM = 1000

_pc = pl.pallas_call


_SC_CH = 128


def _sc_gather_body(nchunks, tail, d, table_hbm, idx_hbm, out_hbm,
                    idx_v, rows_v, idx_t, rows_t, sem):
    info = plsc.get_sparse_core_info()
    nw = info.num_cores * info.num_subcores
    span = idx_hbm.shape[0] // nw
    wid = lax.axis_index("s") * info.num_cores + lax.axis_index("c")
    base = wid * span

    def chunk(j, _):
        off = base + j * _SC_CH
        pltpu.sync_copy(idx_hbm.at[pl.ds(off, _SC_CH)], idx_v)
        pltpu.async_copy(table_hbm.at[idx_v], rows_v, sem).wait()
        pltpu.sync_copy(rows_v, out_hbm.at[pl.ds(off, _SC_CH)])
        return ()

    lax.fori_loop(0, nchunks, chunk, ())
    if tail:
        off = base + nchunks * _SC_CH
        pltpu.sync_copy(idx_hbm.at[pl.ds(off, tail)], idx_t)
        pltpu.async_copy(table_hbm.at[idx_t], rows_t, sem).wait()
        pltpu.sync_copy(rows_t, out_hbm.at[pl.ds(off, tail)])


def _sc_gather(table, idx):
    v, d = table.shape
    e = idx.shape[0]
    info = plsc.get_sparse_core_info()
    nw = info.num_cores * info.num_subcores
    span = e // nw
    assert span * nw == e and span % 8 == 0
    nchunks = span // _SC_CH
    tail = span - nchunks * _SC_CH
    mesh = plsc.VectorSubcoreMesh(core_axis_name="c", subcore_axis_name="s")
    fn = functools.partial(_sc_gather_body, nchunks, tail, d)
    k = pl.kernel(
        fn,
        mesh=mesh,
        out_type=jax.ShapeDtypeStruct((e, d), jnp.float32),
        scratch_types=[
            pltpu.VMEM((_SC_CH,), jnp.int32),
            pltpu.VMEM((_SC_CH, d), jnp.float32),
            pltpu.VMEM((max(tail, 8),), jnp.int32),
            pltpu.VMEM((max(tail, 8), d), jnp.float32),
            pltpu.SemaphoreType.DMA,
        ],
    )
    return k(table, idx.astype(jnp.int32))


def _mm_body(x_ref, w_ref, b_ref, o_ref):
    o_ref[...] = jnp.dot(x_ref[...], w_ref[...],
                         preferred_element_type=jnp.float32) + b_ref[...]


def _dense(xa, w, b, bm=BM):
    m, k = xa.shape
    n = w.shape[1]
    return _pc(
        _mm_body,
        grid=(m // bm,),
        in_specs=[pl.BlockSpec((bm, k), lambda i: (i, 0)),
                  pl.BlockSpec((k, n), lambda i: (0, 0)),
                  pl.BlockSpec((1, n), lambda i: (0, 0))],
        out_specs=pl.BlockSpec((bm, n), lambda i: (i, 0)),
        out_shape=jax.ShapeDtypeStruct((m, n), jnp.float32),
    )(xa, w, b.reshape(1, n))


def _upd_body(agg_ref, sw_ref, h0_ref, g_ref, bz_ref, o_ref):
    out = (1.0 - ALPHA) * agg_ref[...] + sw_ref[...] + ALPHA * h0_ref[...]
    mu = jnp.mean(out, axis=1, keepdims=True)
    var = jnp.mean((out - mu) ** 2, axis=1, keepdims=True)
    out = (out - mu) / jnp.sqrt(var + 1e-5) * g_ref[...] + bz_ref[...]
    o_ref[...] = 0.5 * out * (1.0 + jax.lax.erf(out * 0.7071067811865475))


def _update(agg, sw, h0, g, bz):
    return _pc(
        _upd_body,
        grid=(N // BM,),
        in_specs=[pl.BlockSpec((BM, H), lambda i: (i, 0)),
                  pl.BlockSpec((BM, H), lambda i: (i, 0)),
                  pl.BlockSpec((BM, H), lambda i: (i, 0)),
                  pl.BlockSpec((1, H), lambda i: (0, 0)),
                  pl.BlockSpec((1, H), lambda i: (0, 0))],
        out_specs=pl.BlockSpec((BM, H), lambda i: (i, 0)),
        out_shape=jax.ShapeDtypeStruct((N, H), jnp.float32),
    )(agg, sw, h0, g.reshape(1, H), bz.reshape(1, H))


def _gru_step(xt, h, wih_t, whh_t, bih, bhh):
    gi = jnp.dot(xt, wih_t, preferred_element_type=jnp.float32) + bih
    gh = jnp.dot(h, whh_t, preferred_element_type=jnp.float32) + bhh
    r = jax.nn.sigmoid(gi[:, :H] + gh[:, :H])
    z = jax.nn.sigmoid(gi[:, H:2 * H] + gh[:, H:2 * H])
    n = jnp.tanh(gi[:, 2 * H:] + r * gh[:, 2 * H:])
    return (1.0 - z) * n + z * h


def _seq_body(xs_ref, wihf_ref, whhf_ref, bihf_ref, bhhf_ref,
              wihb_ref, whhb_ref, bihb_ref, bhhb_ref,
              s1_ref, sb1_ref, s2_ref, sb2_ref, s3_ref, sb3_ref, o_ref):
    wihf = wihf_ref[...]
    whhf = whhf_ref[...]
    bihf = bihf_ref[...]
    bhhf = bhhf_ref[...]
    wihb = wihb_ref[...]
    whhb = whhb_ref[...]
    bihb = bihb_ref[...]
    bhhb = bhhb_ref[...]
    hf = jnp.zeros((B, H), dtype=jnp.float32)
    hb = jnp.zeros((B, H), dtype=jnp.float32)
    fs = [None] * T
    bs_ = [None] * T
    for i in range(T):
        hf = _gru_step(xs_ref[i], hf, wihf, whhf, bihf, bhhf)
        fs[i] = hf
        hb = _gru_step(xs_ref[T - 1 - i], hb, wihb, whhb, bihb, bhhb)
        bs_[T - 1 - i] = hb
    acc = jnp.zeros((B, 2 * H), dtype=jnp.float32)
    mx = jnp.full((B, 2 * H), -jnp.inf, dtype=jnp.float32)
    for t in range(T):
        so_t = jnp.concatenate([fs[t], bs_[t]], axis=1)
        acc = acc + so_t
        mx = jnp.maximum(mx, so_t)
    sf = acc * (1.0 / T) + mx
    sf = jnp.maximum(jnp.dot(sf, s1_ref[...], preferred_element_type=jnp.float32)
                     + sb1_ref[...], 0.0)
    sf = jnp.maximum(jnp.dot(sf, s2_ref[...], preferred_element_type=jnp.float32)
                     + sb2_ref[...], 0.0)
    o_ref[...] = jnp.dot(sf, s3_ref[...], preferred_element_type=jnp.float32) + sb3_ref[...]


def _seq_branch(sequence, gru_Wih, gru_Whh, gru_bih, gru_bhh,
                S1, Sb1, S2, Sb2, S3, Sb3):
    xs = jnp.transpose(sequence.reshape(B, T, SEQ_IN), (1, 0, 2))
    args = (xs,
            gru_Wih[0].T, gru_Whh[0].T, gru_bih[0].reshape(1, 3 * H),
            gru_bhh[0].reshape(1, 3 * H),
            gru_Wih[1].T, gru_Whh[1].T, gru_bih[1].reshape(1, 3 * H),
            gru_bhh[1].reshape(1, 3 * H),
            S1, Sb1.reshape(1, H), S2, Sb2.reshape(1, H // 2),
            S3, Sb3.reshape(1, NC))
    specs = [pl.BlockSpec(a.shape, lambda i, _n=len(a.shape): (0,) * _n)
             for a in args]
    return _pc(
        _seq_body,
        grid=(1,),
        in_specs=specs,
        out_specs=pl.BlockSpec((B, NC), lambda i: (0, 0)),
        out_shape=jax.ShapeDtypeStruct((B, NC), jnp.float32),
    )(*args)


def _pool_body(h_ref, bt_ref, g1_ref, gb1_ref, g2_ref, gb2_ref,
               g3_ref, gb3_ref, sl_ref, o_ref):
    iota_b = jax.lax.broadcasted_iota(jnp.int32, (B, N), 0)
    oh = (iota_b == bt_ref[...]).astype(jnp.float32)
    pooled = jnp.dot(oh, h_ref[...], preferred_element_type=jnp.float32)
    cnt = jnp.sum(oh, axis=1, keepdims=True)
    pooled = pooled / jnp.maximum(cnt, 1.0)
    gf = jnp.maximum(jnp.dot(pooled, g1_ref[...], preferred_element_type=jnp.float32)
                     + gb1_ref[...], 0.0)
    gf = jnp.maximum(jnp.dot(gf, g2_ref[...], preferred_element_type=jnp.float32)
                     + gb2_ref[...], 0.0)
    o_ref[...] = (jnp.dot(gf, g3_ref[...], preferred_element_type=jnp.float32)
                  + gb3_ref[...] + sl_ref[...])


def _pool_head(h, batch, G1, Gb1, G2, Gb2, G3, Gb3, seq_logits):
    args = (h, batch.reshape(1, N).astype(jnp.int32),
            G1, Gb1.reshape(1, H // 2), G2, Gb2.reshape(1, H // 4),
            G3, Gb3.reshape(1, NC), seq_logits)
    specs = [pl.BlockSpec(a.shape, lambda i, _n=len(a.shape): (0,) * _n)
             for a in args]
    return _pc(
        _pool_body,
        grid=(1,),
        in_specs=specs,
        out_specs=pl.BlockSpec((B, NC), lambda i: (0, 0)),
        out_shape=jax.ShapeDtypeStruct((B, NC), jnp.float32),
    )(*args)


def kernel(x, edge_index, edge_type, sequence, batch, Wp, bp, Wr, Ws, bs,
           A1, b1, A2, b2, Eemb, imp, ln_g, ln_b,
           gru_Wih, gru_Whh, gru_bih, gru_bhh,
           G1, Gb1, G2, Gb2, G3, Gb3, S1, Sb1, S2, Sb2, S3, Sb3):
    src = edge_index[0]
    dst = edge_index[1]
    et = edge_type.astype(jnp.int32)

    seq_logits = _seq_branch(sequence, gru_Wih, gru_Whh, gru_bih, gru_bhh,
                             S1, Sb1, S2, Sb2, S3, Sb3)

    h = _dense(x, Wp, bp)
    h0 = h

    ew_all = jax.nn.softmax(imp, axis=-1)
    ce_all = jnp.mean(Eemb, axis=2)

    for l in range(L):
        wcat = jnp.concatenate([
            jnp.transpose(jnp.concatenate([A1[l, :, H:, :], Wr[l]], axis=2),
                          (1, 0, 2)).reshape(H, R * 384),
            jnp.transpose(A1[l, :, :H, :], (1, 0, 2)).reshape(H, R * (H // 2)),
            Ws[l],
        ], axis=1)
        bcat = jnp.concatenate([
            jnp.zeros((R * 384,), jnp.float32),
            b1[l].reshape(R * (H // 2)),
            bs[l],
        ])
        o = _dense(h, wcat, bcat)
        hstx = o[:, :R * 384].reshape(N * R, 384)
        hdf = o[:, R * 384:R * 384 + R * (H // 2)].reshape(N * R, H // 2)
        sw = o[:, R * 384 + R * (H // 2):]

        idx_d = dst * R + et
        idx_s = src * R + et
        g_st = _sc_gather(hstx, idx_s)
        g_d = _sc_gather(hdf, idx_d)
        pre = jnp.maximum(g_d + g_st[:, :H // 2], 0.0)
        oh = (et[:, None] == jnp.arange(R, dtype=jnp.int32)).astype(jnp.float32)
        a2e = oh @ A2[l, :, :, 0]
        sc = jnp.sum(pre * a2e, axis=1) + oh @ b2[l, :, 0]
        sc = jnp.where(sc >= 0.0, sc, 0.2 * sc) + oh @ ce_all[l]
        mxs = jnp.max(jnp.where(oh > 0.0, sc[:, None], -1e30), axis=0)
        es = jnp.exp(sc - oh @ mxs)
        dn = es @ oh
        w = es / (oh @ dn) * (oh @ ew_all[l])
        msg = w[:, None] * g_st[:, H // 2:]
        agg = jnp.zeros((N, H), jnp.float32).at[dst].add(msg)

        h = _update(agg, sw, h0, ln_g[l], ln_b[l])

    return _pool_head(h, batch, G1, Gb1, G2, Gb2, G3, Gb3, seq_logits)


# speedup vs baseline: 6.0234x; 1.1165x over previous
import functools

import jax
import jax.numpy as jnp
from jax import lax
from jax.experimental import pallas as pl
from jax.experimental.pallas import tpu as pltpu
from jax.experimental.pallas import tpu_sc as plsc

N = 10000
E = 160000
B = 64
T = 6
DIN = 768
H = 256
SEQ_IN = 128
NC = 14
R = 4
L = 3
ALPHA = 0.15
BM = 1000

_pc = pl.pallas_call


_SC_CH = 128


def _sc_gather_body(nchunks, tail, d, table_hbm, idx_hbm, out_hbm,
                    idx_v, rows_v, idx_t, rows_t, sem):
    info = plsc.get_sparse_core_info()
    nw = info.num_cores * info.num_subcores
    span = idx_hbm.shape[0] // nw
    wid = lax.axis_index("s") * info.num_cores + lax.axis_index("c")
    base = wid * span

    def chunk(j, _):
        off = base + j * _SC_CH
        pltpu.sync_copy(idx_hbm.at[pl.ds(off, _SC_CH)], idx_v)
        pltpu.async_copy(table_hbm.at[idx_v], rows_v, sem).wait()
        pltpu.sync_copy(rows_v, out_hbm.at[pl.ds(off, _SC_CH)])
        return ()

    lax.fori_loop(0, nchunks, chunk, ())
    if tail:
        off = base + nchunks * _SC_CH
        pltpu.sync_copy(idx_hbm.at[pl.ds(off, tail)], idx_t)
        pltpu.async_copy(table_hbm.at[idx_t], rows_t, sem).wait()
        pltpu.sync_copy(rows_t, out_hbm.at[pl.ds(off, tail)])


def _sc_gather(table, idx):
    v, d = table.shape
    e = idx.shape[0]
    info = plsc.get_sparse_core_info()
    nw = info.num_cores * info.num_subcores
    span = e // nw
    assert span * nw == e and span % 8 == 0
    nchunks = span // _SC_CH
    tail = span - nchunks * _SC_CH
    mesh = plsc.VectorSubcoreMesh(core_axis_name="c", subcore_axis_name="s")
    fn = functools.partial(_sc_gather_body, nchunks, tail, d)
    k = pl.kernel(
        fn,
        mesh=mesh,
        out_type=jax.ShapeDtypeStruct((e, d), jnp.float32),
        scratch_types=[
            pltpu.VMEM((_SC_CH,), jnp.int32),
            pltpu.VMEM((_SC_CH, d), jnp.float32),
            pltpu.VMEM((max(tail, 8),), jnp.int32),
            pltpu.VMEM((max(tail, 8), d), jnp.float32),
            pltpu.SemaphoreType.DMA,
        ],
    )
    return k(table, idx.astype(jnp.int32))


def _mm_body(x_ref, w_ref, b_ref, o_ref):
    o_ref[...] = jnp.dot(x_ref[...], w_ref[...],
                         preferred_element_type=jnp.float32) + b_ref[...]


def _dense(xa, w, b, bm=BM):
    m, k = xa.shape
    n = w.shape[1]
    return _pc(
        _mm_body,
        grid=(m // bm,),
        in_specs=[pl.BlockSpec((bm, k), lambda i: (i, 0)),
                  pl.BlockSpec((k, n), lambda i: (0, 0)),
                  pl.BlockSpec((1, n), lambda i: (0, 0))],
        out_specs=pl.BlockSpec((bm, n), lambda i: (i, 0)),
        out_shape=jax.ShapeDtypeStruct((m, n), jnp.float32),
    )(xa, w, b.reshape(1, n))


def _upd_body(agg_ref, sw_ref, h0_ref, g_ref, bz_ref, o_ref):
    out = (1.0 - ALPHA) * agg_ref[...] + sw_ref[...] + ALPHA * h0_ref[...]
    mu = jnp.mean(out, axis=1, keepdims=True)
    var = jnp.mean((out - mu) ** 2, axis=1, keepdims=True)
    out = (out - mu) / jnp.sqrt(var + 1e-5) * g_ref[...] + bz_ref[...]
    o_ref[...] = 0.5 * out * (1.0 + jax.lax.erf(out * 0.7071067811865475))


def _update(agg, sw, h0, g, bz):
    return _pc(
        _upd_body,
        grid=(N // BM,),
        in_specs=[pl.BlockSpec((BM, H), lambda i: (i, 0)),
                  pl.BlockSpec((BM, H), lambda i: (i, 0)),
                  pl.BlockSpec((BM, H), lambda i: (i, 0)),
                  pl.BlockSpec((1, H), lambda i: (0, 0)),
                  pl.BlockSpec((1, H), lambda i: (0, 0))],
        out_specs=pl.BlockSpec((BM, H), lambda i: (i, 0)),
        out_shape=jax.ShapeDtypeStruct((N, H), jnp.float32),
    )(agg, sw, h0, g.reshape(1, H), bz.reshape(1, H))


def _gru_step(xt, h, wih_t, whh_t, bih, bhh):
    gi = jnp.dot(xt, wih_t, preferred_element_type=jnp.float32) + bih
    gh = jnp.dot(h, whh_t, preferred_element_type=jnp.float32) + bhh
    r = jax.nn.sigmoid(gi[:, :H] + gh[:, :H])
    z = jax.nn.sigmoid(gi[:, H:2 * H] + gh[:, H:2 * H])
    n = jnp.tanh(gi[:, 2 * H:] + r * gh[:, 2 * H:])
    return (1.0 - z) * n + z * h


def _seq_body(xs_ref, wihf_ref, whhf_ref, bihf_ref, bhhf_ref,
              wihb_ref, whhb_ref, bihb_ref, bhhb_ref,
              s1_ref, sb1_ref, s2_ref, sb2_ref, s3_ref, sb3_ref, o_ref):
    wihf = wihf_ref[...]
    whhf = whhf_ref[...]
    bihf = bihf_ref[...]
    bhhf = bhhf_ref[...]
    wihb = wihb_ref[...]
    whhb = whhb_ref[...]
    bihb = bihb_ref[...]
    bhhb = bhhb_ref[...]
    hf = jnp.zeros((B, H), dtype=jnp.float32)
    hb = jnp.zeros((B, H), dtype=jnp.float32)
    fs = [None] * T
    bs_ = [None] * T
    for i in range(T):
        hf = _gru_step(xs_ref[i], hf, wihf, whhf, bihf, bhhf)
        fs[i] = hf
        hb = _gru_step(xs_ref[T - 1 - i], hb, wihb, whhb, bihb, bhhb)
        bs_[T - 1 - i] = hb
    acc = jnp.zeros((B, 2 * H), dtype=jnp.float32)
    mx = jnp.full((B, 2 * H), -jnp.inf, dtype=jnp.float32)
    for t in range(T):
        so_t = jnp.concatenate([fs[t], bs_[t]], axis=1)
        acc = acc + so_t
        mx = jnp.maximum(mx, so_t)
    sf = acc * (1.0 / T) + mx
    sf = jnp.maximum(jnp.dot(sf, s1_ref[...], preferred_element_type=jnp.float32)
                     + sb1_ref[...], 0.0)
    sf = jnp.maximum(jnp.dot(sf, s2_ref[...], preferred_element_type=jnp.float32)
                     + sb2_ref[...], 0.0)
    o_ref[...] = jnp.dot(sf, s3_ref[...], preferred_element_type=jnp.float32) + sb3_ref[...]


def _seq_branch(sequence, gru_Wih, gru_Whh, gru_bih, gru_bhh,
                S1, Sb1, S2, Sb2, S3, Sb3):
    xs = jnp.transpose(sequence.reshape(B, T, SEQ_IN), (1, 0, 2))
    args = (xs,
            gru_Wih[0].T, gru_Whh[0].T, gru_bih[0].reshape(1, 3 * H),
            gru_bhh[0].reshape(1, 3 * H),
            gru_Wih[1].T, gru_Whh[1].T, gru_bih[1].reshape(1, 3 * H),
            gru_bhh[1].reshape(1, 3 * H),
            S1, Sb1.reshape(1, H), S2, Sb2.reshape(1, H // 2),
            S3, Sb3.reshape(1, NC))
    specs = [pl.BlockSpec(a.shape, lambda i, _n=len(a.shape): (0,) * _n)
             for a in args]
    return _pc(
        _seq_body,
        grid=(1,),
        in_specs=specs,
        out_specs=pl.BlockSpec((B, NC), lambda i: (0, 0)),
        out_shape=jax.ShapeDtypeStruct((B, NC), jnp.float32),
    )(*args)


def _pool_body(h_ref, bt_ref, g1_ref, gb1_ref, g2_ref, gb2_ref,
               g3_ref, gb3_ref, sl_ref, o_ref):
    iota_b = jax.lax.broadcasted_iota(jnp.int32, (B, N), 0)
    oh = (iota_b == bt_ref[...]).astype(jnp.float32)
    pooled = jnp.dot(oh, h_ref[...], preferred_element_type=jnp.float32)
    cnt = jnp.sum(oh, axis=1, keepdims=True)
    pooled = pooled / jnp.maximum(cnt, 1.0)
    gf = jnp.maximum(jnp.dot(pooled, g1_ref[...], preferred_element_type=jnp.float32)
                     + gb1_ref[...], 0.0)
    gf = jnp.maximum(jnp.dot(gf, g2_ref[...], preferred_element_type=jnp.float32)
                     + gb2_ref[...], 0.0)
    o_ref[...] = (jnp.dot(gf, g3_ref[...], preferred_element_type=jnp.float32)
                  + gb3_ref[...] + sl_ref[...])


def _pool_head(h, batch, G1, Gb1, G2, Gb2, G3, Gb3, seq_logits):
    args = (h, batch.reshape(1, N).astype(jnp.int32),
            G1, Gb1.reshape(1, H // 2), G2, Gb2.reshape(1, H // 4),
            G3, Gb3.reshape(1, NC), seq_logits)
    specs = [pl.BlockSpec(a.shape, lambda i, _n=len(a.shape): (0,) * _n)
             for a in args]
    return _pc(
        _pool_body,
        grid=(1,),
        in_specs=specs,
        out_specs=pl.BlockSpec((B, NC), lambda i: (0, 0)),
        out_shape=jax.ShapeDtypeStruct((B, NC), jnp.float32),
    )(*args)


def kernel(x, edge_index, edge_type, sequence, batch, Wp, bp, Wr, Ws, bs,
           A1, b1, A2, b2, Eemb, imp, ln_g, ln_b,
           gru_Wih, gru_Whh, gru_bih, gru_bhh,
           G1, Gb1, G2, Gb2, G3, Gb3, S1, Sb1, S2, Sb2, S3, Sb3):
    src = edge_index[0]
    dst = edge_index[1]
    et = edge_type.astype(jnp.int32)

    seq_logits = _seq_branch(sequence, gru_Wih, gru_Whh, gru_bih, gru_bhh,
                             S1, Sb1, S2, Sb2, S3, Sb3)

    h = _dense(x, Wp, bp)
    h0 = h

    ew_all = jax.nn.softmax(imp, axis=-1)
    ce_all = jnp.mean(Eemb, axis=2)

    for l in range(L):
        wcat = jnp.concatenate([
            jnp.transpose(jnp.concatenate([A1[l, :, H:, :], Wr[l]], axis=2),
                          (1, 0, 2)).reshape(H, R * 384),
            jnp.transpose(A1[l, :, :H, :], (1, 0, 2)).reshape(H, R * (H // 2)),
            Ws[l],
        ], axis=1)
        bcat = jnp.concatenate([
            jnp.zeros((R * 384,), jnp.float32),
            b1[l].reshape(R * (H // 2)),
            bs[l],
        ])
        o = _dense(h, wcat, bcat)
        hstx = o[:, :R * 384].reshape(N * R, 384)
        hdf = o[:, R * 384:R * 384 + R * (H // 2)].reshape(N * R, H // 2)
        sw = o[:, R * 384 + R * (H // 2):]

        idx_d = dst * R + et
        idx_s = src * R + et
        g_st = _sc_gather(hstx, idx_s)
        g_d = _sc_gather(hdf, idx_d)
        pre = jnp.maximum(g_d + g_st[:, :H // 2], 0.0)
        oh = (et[:, None] == jnp.arange(R, dtype=jnp.int32)).astype(jnp.float32)
        a2e = oh @ A2[l, :, :, 0]
        sc = jnp.sum(pre * a2e, axis=1) + oh @ b2[l, :, 0]
        sc = jnp.where(sc >= 0.0, sc, 0.2 * sc) + oh @ ce_all[l]
        mxs = jnp.max(jnp.where(oh > 0.0, sc[:, None], -1e30), axis=0)
        es = jnp.exp(sc - oh @ mxs)
        dn = es @ oh
        w = es / (oh @ dn) * (oh @ ew_all[l])
        msg = (w[:, None] * g_st[:, H // 2:]).astype(jnp.bfloat16)
        agg = jnp.zeros((N, H), jnp.bfloat16).at[dst].add(msg).astype(jnp.float32)

        h = _update(agg, sw, h0, ln_g[l], ln_b[l])

    return _pool_head(h, batch, G1, Gb1, G2, Gb2, G3, Gb3, seq_logits)
